# Initial kernel scaffold; baseline (speedup 1.0000x reference)
import jax
import jax.numpy as jnp
from jax.experimental import pallas as pl


def kernel(hidden_states, gate_weight, gate_up_weights, down_weights):
    raise NotImplementedError("write your pallas kernel here")


# speedup vs baseline: 1.5231x; 1.5231x over previous
import functools

import jax
import jax.numpy as jnp
from jax import lax
from jax.experimental import pallas as pl
from jax.experimental.pallas import tpu as pltpu
from jax.experimental.pallas import tpu_sc as plsc

D = 2048
I = 1408
E = 8
T = 4096
A = T * 2

TM = 256
M = A // TM
NV = M + E - 1
KT = 1024
NK = D // KT

NW = 32
TPW = T // NW
CH = 16
NCH = TPW // CH


def _cumsum_lanes(a, n):
    s = 1
    while s < n:
        a = a + jnp.concatenate(
            [jnp.zeros((a.shape[0], s), a.dtype), a[:, : n - s]], axis=1)
        s *= 2
    return a


def _router_kernel(x_ref, gw_ref, dest_ref, w_ref, offs_ref, logits_scr):
    step = pl.program_id(0)
    nb = pl.num_programs(0)
    xb = x_ref[...]
    lg = lax.dot_general(gw_ref[...], xb, (((1,), (1,)), ((), ())),
                         preferred_element_type=jnp.float32)
    tb = xb.shape[0]
    logits_scr[:, pl.ds(step * tb, tb)] = lg

    @pl.when(step == nb - 1)
    def _():
        lt = logits_scr[...]
        m = jnp.max(lt, axis=0, keepdims=True)
        exl = jnp.exp(lt - m)
        p = exl / jnp.sum(exl, axis=0, keepdims=True)
        ei = lax.broadcasted_iota(jnp.int32, (E, T), 0)
        m0 = jnp.max(p, axis=0, keepdims=True)
        e0 = jnp.min(jnp.where(p == m0, ei, E), axis=0, keepdims=True)
        is0 = ei == e0
        p2 = jnp.where(is0, -1.0, p)
        m1 = jnp.max(p2, axis=0, keepdims=True)
        e1 = jnp.min(jnp.where(p2 == m1, ei, E), axis=0, keepdims=True)
        is1 = ei == e1
        s = m0 + m1
        s = jnp.where(s <= 0.0, 1.0, s)
        oh0 = is0.astype(jnp.int32)
        oh1 = is1.astype(jnp.int32)
        c0 = _cumsum_lanes(oh0, T)
        cnt0 = c0[:, T - 1 : T]
        c1 = _cumsum_lanes(oh1, T) + cnt0
        cnt = c1[:, T - 1 : T]
        inc = cnt
        st = 1
        while st < E:
            inc = inc + jnp.concatenate(
                [jnp.zeros((st, 1), inc.dtype), inc[: E - st, :]], axis=0)
            st *= 2
        off = inc - cnt
        dest_ref[0:1, :] = jnp.sum(oh0 * (off + c0 - 1), axis=0, keepdims=True)
        dest_ref[1:2, :] = jnp.sum(oh1 * (off + c1 - 1), axis=0, keepdims=True)
        w_ref[0:1, :] = m0 / s
        w_ref[1:2, :] = m1 / s
        offs_ref[...] = off


def _route(x, gate_weight, interpret=False):
    tb = 512
    return pl.pallas_call(
        _router_kernel,
        grid=(T // tb,),
        in_specs=[
            pl.BlockSpec((tb, D), lambda s: (s, 0)),
            pl.BlockSpec((E, D), lambda s: (0, 0)),
        ],
        out_specs=[
            pl.BlockSpec((2, T), lambda s: (0, 0)),
            pl.BlockSpec((2, T), lambda s: (0, 0)),
            pl.BlockSpec((E, 1), lambda s: (0, 0)),
        ],
        out_shape=[
            jax.ShapeDtypeStruct((2, T), jnp.int32),
            jax.ShapeDtypeStruct((2, T), jnp.float32),
            jax.ShapeDtypeStruct((E, 1), jnp.int32),
        ],
        scratch_shapes=[pltpu.VMEM((E, T), jnp.float32)],
        interpret=interpret,
    )(x, gate_weight)


def _tile_metadata(off):
    off_e = off[:, 0]
    seg_lo = off_e
    seg_hi = jnp.concatenate([off_e[1:], jnp.array([A], jnp.int32)])
    mgrid = jnp.arange(M, dtype=jnp.int32)[:, None]
    st = jnp.maximum(seg_lo[None, :], mgrid * TM)
    en = jnp.minimum(seg_hi[None, :], mgrid * TM + TM)
    needed = en > st
    flat = needed.reshape(-1)
    rank = jnp.cumsum(flat) - flat.astype(jnp.int32)
    vidx = jnp.where(flat, rank, NV)
    inrow = jnp.cumsum(needed, axis=1) - needed.astype(jnp.int32)
    first = (needed & (inrow == 0)).reshape(-1).astype(jnp.int32)
    mt = jnp.full((NV,), M - 1, jnp.int32).at[vidx].set(
        jnp.broadcast_to(mgrid, (M, E)).reshape(-1), mode="drop")
    gid = jnp.zeros((NV,), jnp.int32).at[vidx].set(
        jnp.broadcast_to(jnp.arange(E, dtype=jnp.int32)[None, :],
                         (M, E)).reshape(-1), mode="drop")
    rs = jnp.zeros((NV,), jnp.int32).at[vidx].set(st.reshape(-1), mode="drop")
    re_ = jnp.zeros((NV,), jnp.int32).at[vidx].set(en.reshape(-1), mode="drop")
    fs = jnp.zeros((NV,), jnp.int32).at[vidx].set(first, mode="drop")
    return mt, gid, rs, re_, fs


def _gmm_kernel(mt, gid, rs, re_, fs,
                x_ref, g_ref, u_ref, d_ref, o_ref, g_scr, u_scr):
    v = pl.program_id(0)
    k = pl.program_id(1)
    xb = x_ref[...]
    pg = jnp.dot(xb, g_ref[0], preferred_element_type=jnp.float32)
    pu = jnp.dot(xb, u_ref[0], preferred_element_type=jnp.float32)

    @pl.when(k == 0)
    def _():
        g_scr[...] = pg
        u_scr[...] = pu

    @pl.when(k == NK - 1)
    def _():
        g = g_scr[...] + pg if NK > 1 else pg
        u = u_scr[...] + pu if NK > 1 else pu
        act = g * jax.nn.sigmoid(g) * u
        r = mt[v] * TM + lax.broadcasted_iota(jnp.int32, (TM, 1), 0)
        rm = (r >= rs[v]) & (r < re_[v])
        act = jnp.where(rm, act, 0.0)
        contrib = jnp.dot(act, d_ref[0], preferred_element_type=jnp.float32)

        @pl.when(fs[v] == 1)
        def _():
            o_ref[...] = contrib

        @pl.when(fs[v] != 1)
        def _():
            o_ref[...] += contrib

    @pl.when((k != 0) & (k != NK - 1))
    def _():
        g_scr[...] += pg
        u_scr[...] += pu


def _gmm(x_sorted, gate_up_w, down_w, meta, interpret=False):
    grid_spec = pltpu.PrefetchScalarGridSpec(
        num_scalar_prefetch=5,
        grid=(NV, NK),
        in_specs=[
            pl.BlockSpec((TM, KT), lambda v, k, mt, gid, rs, re_, fs: (mt[v], k)),
            pl.BlockSpec((1, KT, I),
                         lambda v, k, mt, gid, rs, re_, fs: (gid[v], k, 0)),
            pl.BlockSpec((1, KT, I),
                         lambda v, k, mt, gid, rs, re_, fs: (gid[v], k, 1)),
            pl.BlockSpec((1, I, D),
                         lambda v, k, mt, gid, rs, re_, fs: (gid[v], 0, 0)),
        ],
        out_specs=pl.BlockSpec((TM, D),
                               lambda v, k, mt, gid, rs, re_, fs: (mt[v], 0)),
        scratch_shapes=[pltpu.VMEM((TM, I), jnp.float32),
                        pltpu.VMEM((TM, I), jnp.float32)],
    )
    return pl.pallas_call(
        _gmm_kernel,
        grid_spec=grid_spec,
        out_shape=jax.ShapeDtypeStruct((A, D), jnp.float32),
        interpret=interpret,
    )(*meta, x_sorted, gate_up_w, gate_up_w, down_w)


def _sc_dispatch(x, dest):
    mesh = plsc.VectorSubcoreMesh(core_axis_name="c", subcore_axis_name="s")

    @functools.partial(
        pl.kernel, mesh=mesh,
        out_type=jax.ShapeDtypeStruct((A, D), jnp.float32),
        scratch_types=[
            pltpu.VMEM((2, TPW), jnp.int32),
            pltpu.VMEM((CH, D), jnp.float32),
            pltpu.SemaphoreType.DMA,
        ],
    )
    def k(x_hbm, dest_hbm, xs_hbm, d_v, xbuf, sem):
        cid = lax.axis_index("c")
        sid = lax.axis_index("s")
        wid = sid * 2 + cid
        base = wid * TPW
        pltpu.sync_copy(dest_hbm.at[:, pl.ds(base, TPW)], d_v)
        for c in range(NCH):
            pltpu.sync_copy(x_hbm.at[pl.ds(base + c * CH, CH)], xbuf)
            i0 = d_v[0, pl.ds(c * CH, CH)]
            pltpu.async_copy(xbuf, xs_hbm.at[i0], sem).wait()
            i1 = d_v[1, pl.ds(c * CH, CH)]
            pltpu.async_copy(xbuf, xs_hbm.at[i1], sem).wait()

    return k(x, dest)


def _sc_combine(y_sorted, dest, w):
    mesh = plsc.VectorSubcoreMesh(core_axis_name="c", subcore_axis_name="s")

    @functools.partial(
        pl.kernel, mesh=mesh,
        out_type=jax.ShapeDtypeStruct((T, D), jnp.float32),
        scratch_types=[
            pltpu.VMEM((2, TPW), jnp.int32),
            pltpu.VMEM((2, TPW), jnp.float32),
            pltpu.VMEM((CH, D), jnp.float32),
            pltpu.VMEM((CH, D), jnp.float32),
            pltpu.SemaphoreType.DMA,
            pltpu.SemaphoreType.DMA,
        ],
    )
    def k(ys_hbm, dest_hbm, w_hbm, out_hbm, d_v, w_v, b0, b1, sem0, sem1):
        cid = lax.axis_index("c")
        sid = lax.axis_index("s")
        wid = sid * 2 + cid
        base = wid * TPW
        pltpu.sync_copy(dest_hbm.at[:, pl.ds(base, TPW)], d_v)
        pltpu.sync_copy(w_hbm.at[:, pl.ds(base, TPW)], w_v)
        for c in range(NCH):
            i0 = d_v[0, pl.ds(c * CH, CH)]
            i1 = d_v[1, pl.ds(c * CH, CH)]
            cp0 = pltpu.async_copy(ys_hbm.at[i0], b0, sem0)
            cp1 = pltpu.async_copy(ys_hbm.at[i1], b1, sem1)
            cp0.wait()
            cp1.wait()
            wc0 = w_v[0, pl.ds(c * CH, CH)]
            wc1 = w_v[1, pl.ds(c * CH, CH)]
            for rr in range(CH):
                lane = jnp.full((CH,), rr, jnp.int32)
                wb0 = wc0[lane]
                wb1 = wc1[lane]

                def body(j, _):
                    for u in range(4):
                        sl = pl.ds(j * 64 + u * 16, 16)
                        b0[rr, sl] = wb0 * b0[rr, sl] + wb1 * b1[rr, sl]
                    return 0
                lax.fori_loop(0, D // 64, body, 0)
            pltpu.sync_copy(b0, out_hbm.at[pl.ds(base + c * CH, CH)])

    return k(y_sorted, dest, w)


def kernel(hidden_states, gate_weight, gate_up_weights, down_weights):
    x = hidden_states.reshape(-1, D)
    dest, w, off = _route(x, gate_weight)
    meta = _tile_metadata(off)
    x_sorted = _sc_dispatch(x, dest)
    y = _gmm(x_sorted, gate_up_weights, down_weights, meta)
    out = _sc_combine(y, dest, w)
    return out.reshape(hidden_states.shape)


# speedup vs baseline: 1.5671x; 1.0289x over previous
import functools

import jax
import jax.numpy as jnp
from jax import lax
from jax.experimental import pallas as pl
from jax.experimental.pallas import tpu as pltpu
from jax.experimental.pallas import tpu_sc as plsc

D = 2048
I = 1408
E = 8
T = 4096
A = T * 2

TM = 512
M = A // TM
NV = M + E - 1
KT = 2048
NK = D // KT

NW = 32
TPW = T // NW
CH = 16
NCH = TPW // CH


def _cumsum_lanes(a, n):
    s = 1
    while s < n:
        a = a + jnp.concatenate(
            [jnp.zeros((a.shape[0], s), a.dtype), a[:, : n - s]], axis=1)
        s *= 2
    return a


def _router_kernel(x_ref, gw_ref, dest_ref, w_ref, offs_ref, logits_scr):
    step = pl.program_id(0)
    nb = pl.num_programs(0)
    xb = x_ref[...]
    lg = lax.dot_general(gw_ref[...], xb, (((1,), (1,)), ((), ())),
                         preferred_element_type=jnp.float32)
    tb = xb.shape[0]
    logits_scr[:, pl.ds(step * tb, tb)] = lg

    @pl.when(step == nb - 1)
    def _():
        lt = logits_scr[...]
        m = jnp.max(lt, axis=0, keepdims=True)
        exl = jnp.exp(lt - m)
        p = exl / jnp.sum(exl, axis=0, keepdims=True)
        ei = lax.broadcasted_iota(jnp.int32, (E, T), 0)
        m0 = jnp.max(p, axis=0, keepdims=True)
        e0 = jnp.min(jnp.where(p == m0, ei, E), axis=0, keepdims=True)
        is0 = ei == e0
        p2 = jnp.where(is0, -1.0, p)
        m1 = jnp.max(p2, axis=0, keepdims=True)
        e1 = jnp.min(jnp.where(p2 == m1, ei, E), axis=0, keepdims=True)
        is1 = ei == e1
        s = m0 + m1
        s = jnp.where(s <= 0.0, 1.0, s)
        oh0 = is0.astype(jnp.int32)
        oh1 = is1.astype(jnp.int32)
        c0 = _cumsum_lanes(oh0, T)
        cnt0 = c0[:, T - 1 : T]
        c1 = _cumsum_lanes(oh1, T) + cnt0
        cnt = c1[:, T - 1 : T]
        inc = cnt
        st = 1
        while st < E:
            inc = inc + jnp.concatenate(
                [jnp.zeros((st, 1), inc.dtype), inc[: E - st, :]], axis=0)
            st *= 2
        off = inc - cnt
        dest_ref[0:1, :] = jnp.sum(oh0 * (off + c0 - 1), axis=0, keepdims=True)
        dest_ref[1:2, :] = jnp.sum(oh1 * (off + c1 - 1), axis=0, keepdims=True)
        w_ref[0:1, :] = m0 / s
        w_ref[1:2, :] = m1 / s
        offs_ref[...] = off


def _route(x, gate_weight, interpret=False):
    tb = 512
    return pl.pallas_call(
        _router_kernel,
        grid=(T // tb,),
        in_specs=[
            pl.BlockSpec((tb, D), lambda s: (s, 0)),
            pl.BlockSpec((E, D), lambda s: (0, 0)),
        ],
        out_specs=[
            pl.BlockSpec((2, T), lambda s: (0, 0)),
            pl.BlockSpec((2, T), lambda s: (0, 0)),
            pl.BlockSpec((E, 1), lambda s: (0, 0)),
        ],
        out_shape=[
            jax.ShapeDtypeStruct((2, T), jnp.int32),
            jax.ShapeDtypeStruct((2, T), jnp.float32),
            jax.ShapeDtypeStruct((E, 1), jnp.int32),
        ],
        scratch_shapes=[pltpu.VMEM((E, T), jnp.float32)],
        interpret=interpret,
    )(x, gate_weight)


def _tile_metadata(off):
    off_e = off[:, 0]
    seg_lo = off_e
    seg_hi = jnp.concatenate([off_e[1:], jnp.array([A], jnp.int32)])
    mgrid = jnp.arange(M, dtype=jnp.int32)[:, None]
    st = jnp.maximum(seg_lo[None, :], mgrid * TM)
    en = jnp.minimum(seg_hi[None, :], mgrid * TM + TM)
    needed = en > st
    flat = needed.reshape(-1)
    rank = jnp.cumsum(flat) - flat.astype(jnp.int32)
    vidx = jnp.where(flat, rank, NV)
    inrow = jnp.cumsum(needed, axis=1) - needed.astype(jnp.int32)
    first = (needed & (inrow == 0)).reshape(-1).astype(jnp.int32)
    mt = jnp.full((NV,), M - 1, jnp.int32).at[vidx].set(
        jnp.broadcast_to(mgrid, (M, E)).reshape(-1), mode="drop")
    gid = jnp.zeros((NV,), jnp.int32).at[vidx].set(
        jnp.broadcast_to(jnp.arange(E, dtype=jnp.int32)[None, :],
                         (M, E)).reshape(-1), mode="drop")
    rs = jnp.zeros((NV,), jnp.int32).at[vidx].set(st.reshape(-1), mode="drop")
    re_ = jnp.zeros((NV,), jnp.int32).at[vidx].set(en.reshape(-1), mode="drop")
    fs = jnp.zeros((NV,), jnp.int32).at[vidx].set(first, mode="drop")
    return mt, gid, rs, re_, fs


def _gmm_kernel(mt, gid, rs, re_, fs,
                x_ref, g_ref, u_ref, d_ref, o_ref):
    v = pl.program_id(0)
    xb = x_ref[...].astype(jnp.bfloat16)
    g = jnp.dot(xb, g_ref[0], preferred_element_type=jnp.float32)
    u = jnp.dot(xb, u_ref[0], preferred_element_type=jnp.float32)
    act = g * jax.nn.sigmoid(g) * u
    r = mt[v] * TM + lax.broadcasted_iota(jnp.int32, (TM, 1), 0)
    rm = (r >= rs[v]) & (r < re_[v])
    act = jnp.where(rm, act, 0.0).astype(jnp.bfloat16)
    contrib = jnp.dot(act, d_ref[0], preferred_element_type=jnp.float32)

    @pl.when(fs[v] == 1)
    def _():
        o_ref[...] = contrib

    @pl.when(fs[v] != 1)
    def _():
        o_ref[...] += contrib


def _gmm(x_sorted, gate_up_w, down_w, meta, interpret=False):
    grid_spec = pltpu.PrefetchScalarGridSpec(
        num_scalar_prefetch=5,
        grid=(NV,),
        in_specs=[
            pl.BlockSpec((TM, D), lambda v, mt, gid, rs, re_, fs: (mt[v], 0)),
            pl.BlockSpec((1, D, I),
                         lambda v, mt, gid, rs, re_, fs: (gid[v], 0, 0)),
            pl.BlockSpec((1, D, I),
                         lambda v, mt, gid, rs, re_, fs: (gid[v], 0, 1)),
            pl.BlockSpec((1, I, D),
                         lambda v, mt, gid, rs, re_, fs: (gid[v], 0, 0)),
        ],
        out_specs=pl.BlockSpec((TM, D),
                               lambda v, mt, gid, rs, re_, fs: (mt[v], 0)),
    )
    return pl.pallas_call(
        _gmm_kernel,
        grid_spec=grid_spec,
        out_shape=jax.ShapeDtypeStruct((A, D), jnp.float32),
        interpret=interpret,
    )(*meta, x_sorted, gate_up_w, gate_up_w, down_w)


def _sc_dispatch(x, dest):
    mesh = plsc.VectorSubcoreMesh(core_axis_name="c", subcore_axis_name="s")

    @functools.partial(
        pl.kernel, mesh=mesh,
        out_type=jax.ShapeDtypeStruct((A, D), jnp.float32),
        scratch_types=[
            pltpu.VMEM((2, TPW), jnp.int32),
            pltpu.VMEM((CH, D), jnp.float32),
            pltpu.SemaphoreType.DMA,
        ],
    )
    def k(x_hbm, dest_hbm, xs_hbm, d_v, xbuf, sem):
        cid = lax.axis_index("c")
        sid = lax.axis_index("s")
        wid = sid * 2 + cid
        base = wid * TPW
        pltpu.sync_copy(dest_hbm.at[:, pl.ds(base, TPW)], d_v)
        for c in range(NCH):
            pltpu.sync_copy(x_hbm.at[pl.ds(base + c * CH, CH)], xbuf)
            i0 = d_v[0, pl.ds(c * CH, CH)]
            pltpu.async_copy(xbuf, xs_hbm.at[i0], sem).wait()
            i1 = d_v[1, pl.ds(c * CH, CH)]
            pltpu.async_copy(xbuf, xs_hbm.at[i1], sem).wait()

    return k(x, dest)


def _sc_combine(y_sorted, dest, w):
    mesh = plsc.VectorSubcoreMesh(core_axis_name="c", subcore_axis_name="s")

    @functools.partial(
        pl.kernel, mesh=mesh,
        out_type=jax.ShapeDtypeStruct((T, D), jnp.float32),
        scratch_types=[
            pltpu.VMEM((2, TPW), jnp.int32),
            pltpu.VMEM((2, TPW), jnp.float32),
            pltpu.VMEM((CH, D), jnp.float32),
            pltpu.VMEM((CH, D), jnp.float32),
            pltpu.SemaphoreType.DMA,
            pltpu.SemaphoreType.DMA,
        ],
    )
    def k(ys_hbm, dest_hbm, w_hbm, out_hbm, d_v, w_v, b0, b1, sem0, sem1):
        cid = lax.axis_index("c")
        sid = lax.axis_index("s")
        wid = sid * 2 + cid
        base = wid * TPW
        pltpu.sync_copy(dest_hbm.at[:, pl.ds(base, TPW)], d_v)
        pltpu.sync_copy(w_hbm.at[:, pl.ds(base, TPW)], w_v)
        for c in range(NCH):
            i0 = d_v[0, pl.ds(c * CH, CH)]
            i1 = d_v[1, pl.ds(c * CH, CH)]
            cp0 = pltpu.async_copy(ys_hbm.at[i0], b0, sem0)
            cp1 = pltpu.async_copy(ys_hbm.at[i1], b1, sem1)
            cp0.wait()
            cp1.wait()
            wc0 = w_v[0, pl.ds(c * CH, CH)]
            wc1 = w_v[1, pl.ds(c * CH, CH)]
            for rr in range(CH):
                lane = jnp.full((CH,), rr, jnp.int32)
                wb0 = wc0[lane]
                wb1 = wc1[lane]

                def body(j, _):
                    for u in range(4):
                        sl = pl.ds(j * 64 + u * 16, 16)
                        b0[rr, sl] = wb0 * b0[rr, sl] + wb1 * b1[rr, sl]
                    return 0
                lax.fori_loop(0, D // 64, body, 0)
            pltpu.sync_copy(b0, out_hbm.at[pl.ds(base + c * CH, CH)])

    return k(y_sorted, dest, w)


def kernel(hidden_states, gate_weight, gate_up_weights, down_weights):
    x = hidden_states.reshape(-1, D)
    dest, w, off = _route(x, gate_weight)
    meta = _tile_metadata(off)
    x_sorted = _sc_dispatch(x, dest)
    y = _gmm(x_sorted, gate_up_weights.astype(jnp.bfloat16),
             down_weights.astype(jnp.bfloat16), meta)
    out = _sc_combine(y, dest, w)
    return out.reshape(hidden_states.shape)


# speedup vs baseline: 1.6339x; 1.0427x over previous
import functools

import jax
import jax.numpy as jnp
from jax import lax
from jax.experimental import pallas as pl
from jax.experimental.pallas import tpu as pltpu
from jax.experimental.pallas import tpu_sc as plsc

D = 2048
I = 1408
E = 8
T = 4096
A = T * 2

TM = 512
M = A // TM
NV = M + E - 1
KT = 2048
NK = D // KT

NW = 32
TPW = T // NW
CH = 16
NCH = TPW // CH


def _cumsum_lanes(a, n):
    s = 1
    while s < n:
        a = a + jnp.concatenate(
            [jnp.zeros((a.shape[0], s), a.dtype), a[:, : n - s]], axis=1)
        s *= 2
    return a


def _router_kernel(x_ref, gw_ref, dest_ref, w_ref, offs_ref, logits_scr):
    step = pl.program_id(0)
    nb = pl.num_programs(0)
    xb = x_ref[...]
    lg = lax.dot_general(gw_ref[...], xb, (((1,), (1,)), ((), ())),
                         preferred_element_type=jnp.float32)
    tb = xb.shape[0]
    logits_scr[:, pl.ds(step * tb, tb)] = lg

    @pl.when(step == nb - 1)
    def _():
        lt = logits_scr[...]
        m = jnp.max(lt, axis=0, keepdims=True)
        exl = jnp.exp(lt - m)
        p = exl / jnp.sum(exl, axis=0, keepdims=True)
        ei = lax.broadcasted_iota(jnp.int32, (E, T), 0)
        m0 = jnp.max(p, axis=0, keepdims=True)
        e0 = jnp.min(jnp.where(p == m0, ei, E), axis=0, keepdims=True)
        is0 = ei == e0
        p2 = jnp.where(is0, -1.0, p)
        m1 = jnp.max(p2, axis=0, keepdims=True)
        e1 = jnp.min(jnp.where(p2 == m1, ei, E), axis=0, keepdims=True)
        is1 = ei == e1
        s = m0 + m1
        s = jnp.where(s <= 0.0, 1.0, s)
        oh0 = is0.astype(jnp.int32)
        oh1 = is1.astype(jnp.int32)
        c0 = _cumsum_lanes(oh0, T)
        cnt0 = c0[:, T - 1 : T]
        c1 = _cumsum_lanes(oh1, T) + cnt0
        cnt = c1[:, T - 1 : T]
        inc = cnt
        st = 1
        while st < E:
            inc = inc + jnp.concatenate(
                [jnp.zeros((st, 1), inc.dtype), inc[: E - st, :]], axis=0)
            st *= 2
        off = inc - cnt
        dest_ref[0:1, :] = jnp.sum(oh0 * (off + c0 - 1), axis=0, keepdims=True)
        dest_ref[1:2, :] = jnp.sum(oh1 * (off + c1 - 1), axis=0, keepdims=True)
        w_ref[0:1, :] = m0 / s
        w_ref[1:2, :] = m1 / s
        offs_ref[...] = off


def _route(x, gate_weight, interpret=False):
    tb = 512
    return pl.pallas_call(
        _router_kernel,
        grid=(T // tb,),
        in_specs=[
            pl.BlockSpec((tb, D), lambda s: (s, 0)),
            pl.BlockSpec((E, D), lambda s: (0, 0)),
        ],
        out_specs=[
            pl.BlockSpec((2, T), lambda s: (0, 0)),
            pl.BlockSpec((2, T), lambda s: (0, 0)),
            pl.BlockSpec((E, 1), lambda s: (0, 0)),
        ],
        out_shape=[
            jax.ShapeDtypeStruct((2, T), jnp.int32),
            jax.ShapeDtypeStruct((2, T), jnp.float32),
            jax.ShapeDtypeStruct((E, 1), jnp.int32),
        ],
        scratch_shapes=[pltpu.VMEM((E, T), jnp.float32)],
        interpret=interpret,
    )(x, gate_weight)


def _tile_metadata(off):
    off_e = off[:, 0]
    seg_lo = off_e
    seg_hi = jnp.concatenate([off_e[1:], jnp.array([A], jnp.int32)])
    mgrid = jnp.arange(M, dtype=jnp.int32)[None, :]
    st = jnp.maximum(seg_lo[:, None], mgrid * TM)
    en = jnp.minimum(seg_hi[:, None], mgrid * TM + TM)
    needed = en > st
    flat = needed.reshape(-1)
    rank = jnp.cumsum(flat) - flat.astype(jnp.int32)
    vidx = jnp.where(flat, rank, NV)
    incol = jnp.cumsum(needed, axis=0) - needed.astype(jnp.int32)
    first_tile = (needed & (incol == 0)).reshape(-1).astype(jnp.int32)
    inrow = jnp.cumsum(needed, axis=1) - needed.astype(jnp.int32)
    first_of_expert = (needed & (inrow == 0)).reshape(-1).astype(jnp.int32)
    mt = jnp.full((NV,), M - 1, jnp.int32).at[vidx].set(
        jnp.broadcast_to(mgrid, (E, M)).reshape(-1), mode="drop")
    gid = jnp.full((NV,), E - 1, jnp.int32).at[vidx].set(
        jnp.broadcast_to(jnp.arange(E, dtype=jnp.int32)[:, None],
                         (E, M)).reshape(-1), mode="drop")
    rs = jnp.zeros((NV,), jnp.int32).at[vidx].set(st.reshape(-1), mode="drop")
    re_ = jnp.zeros((NV,), jnp.int32).at[vidx].set(en.reshape(-1), mode="drop")
    fs = jnp.zeros((NV,), jnp.int32).at[vidx].set(first_tile, mode="drop")
    enew = jnp.zeros((NV,), jnp.int32).at[vidx].set(first_of_expert,
                                                   mode="drop")
    return mt, gid, rs, re_, fs, enew


NC = 4
DC = D // NC
IC = I // NC


def _gmm_kernel(mt, gid, rs, re_, fs, enew,
                x_ref, gu_ref, dn_ref, o_ref, g_s, u_s, d_s):
    v = pl.program_id(0)
    c = pl.program_id(1)

    @pl.when(enew[v] == 1)
    def _():
        blk = gu_ref[0]
        g_s[pl.ds(c * DC, DC), :] = blk[:, :I].astype(jnp.bfloat16)
        u_s[pl.ds(c * DC, DC), :] = blk[:, I:].astype(jnp.bfloat16)
        d_s[pl.ds(c * IC, IC), :] = dn_ref[0].astype(jnp.bfloat16)

    @pl.when(c == NC - 1)
    def _():
        xb = x_ref[...].astype(jnp.bfloat16)
        g = jnp.dot(xb, g_s[...], preferred_element_type=jnp.float32)
        u = jnp.dot(xb, u_s[...], preferred_element_type=jnp.float32)
        act = g * jax.nn.sigmoid(g) * u
        r = mt[v] * TM + lax.broadcasted_iota(jnp.int32, (TM, 1), 0)
        rm = (r >= rs[v]) & (r < re_[v])
        act = jnp.where(rm, act, 0.0).astype(jnp.bfloat16)
        contrib = jnp.dot(act, d_s[...], preferred_element_type=jnp.float32)

        @pl.when(fs[v] == 1)
        def _():
            o_ref[...] = contrib

        @pl.when(fs[v] != 1)
        def _():
            o_ref[...] += contrib


def _gmm(x_sorted, gate_up_w, down_w, meta, interpret=False):
    def wchunk(c, enew, v):
        return enew[v] * c + (1 - enew[v]) * (NC - 1)

    grid_spec = pltpu.PrefetchScalarGridSpec(
        num_scalar_prefetch=6,
        grid=(NV, NC),
        in_specs=[
            pl.BlockSpec((TM, D),
                         lambda v, c, mt, gid, rs, re_, fs, en: (mt[v], 0)),
            pl.BlockSpec((1, DC, 2 * I),
                         lambda v, c, mt, gid, rs, re_, fs, en:
                         (gid[v], wchunk(c, en, v), 0)),
            pl.BlockSpec((1, IC, D),
                         lambda v, c, mt, gid, rs, re_, fs, en:
                         (gid[v], wchunk(c, en, v), 0)),
        ],
        out_specs=pl.BlockSpec((TM, D),
                               lambda v, c, mt, gid, rs, re_, fs, en:
                               (mt[v], 0)),
        scratch_shapes=[pltpu.VMEM((D, I), jnp.bfloat16),
                        pltpu.VMEM((D, I), jnp.bfloat16),
                        pltpu.VMEM((I, D), jnp.bfloat16)],
    )
    return pl.pallas_call(
        _gmm_kernel,
        grid_spec=grid_spec,
        out_shape=jax.ShapeDtypeStruct((A, D), jnp.float32),
        interpret=interpret,
    )(*meta, x_sorted, gate_up_w, down_w)


def _sc_dispatch(x, dest):
    mesh = plsc.VectorSubcoreMesh(core_axis_name="c", subcore_axis_name="s")

    @functools.partial(
        pl.kernel, mesh=mesh,
        out_type=jax.ShapeDtypeStruct((A, D), jnp.float32),
        scratch_types=[
            pltpu.VMEM((2, TPW), jnp.int32),
            pltpu.VMEM((CH, D), jnp.float32),
            pltpu.SemaphoreType.DMA,
        ],
    )
    def k(x_hbm, dest_hbm, xs_hbm, d_v, xbuf, sem):
        cid = lax.axis_index("c")
        sid = lax.axis_index("s")
        wid = sid * 2 + cid
        base = wid * TPW
        pltpu.sync_copy(dest_hbm.at[:, pl.ds(base, TPW)], d_v)
        for c in range(NCH):
            pltpu.sync_copy(x_hbm.at[pl.ds(base + c * CH, CH)], xbuf)
            i0 = d_v[0, pl.ds(c * CH, CH)]
            pltpu.async_copy(xbuf, xs_hbm.at[i0], sem).wait()
            i1 = d_v[1, pl.ds(c * CH, CH)]
            pltpu.async_copy(xbuf, xs_hbm.at[i1], sem).wait()

    return k(x, dest)


def _sc_combine(y_sorted, dest, w):
    mesh = plsc.VectorSubcoreMesh(core_axis_name="c", subcore_axis_name="s")

    @functools.partial(
        pl.kernel, mesh=mesh,
        out_type=jax.ShapeDtypeStruct((T, D), jnp.float32),
        scratch_types=[
            pltpu.VMEM((2, TPW), jnp.int32),
            pltpu.VMEM((2, TPW), jnp.float32),
            pltpu.VMEM((CH, D), jnp.float32),
            pltpu.VMEM((CH, D), jnp.float32),
            pltpu.SemaphoreType.DMA,
            pltpu.SemaphoreType.DMA,
        ],
    )
    def k(ys_hbm, dest_hbm, w_hbm, out_hbm, d_v, w_v, b0, b1, sem0, sem1):
        cid = lax.axis_index("c")
        sid = lax.axis_index("s")
        wid = sid * 2 + cid
        base = wid * TPW
        pltpu.sync_copy(dest_hbm.at[:, pl.ds(base, TPW)], d_v)
        pltpu.sync_copy(w_hbm.at[:, pl.ds(base, TPW)], w_v)
        for c in range(NCH):
            i0 = d_v[0, pl.ds(c * CH, CH)]
            i1 = d_v[1, pl.ds(c * CH, CH)]
            cp0 = pltpu.async_copy(ys_hbm.at[i0], b0, sem0)
            cp1 = pltpu.async_copy(ys_hbm.at[i1], b1, sem1)
            cp0.wait()
            cp1.wait()
            wc0 = w_v[0, pl.ds(c * CH, CH)]
            wc1 = w_v[1, pl.ds(c * CH, CH)]
            for rr in range(CH):
                lane = jnp.full((CH,), rr, jnp.int32)
                wb0 = wc0[lane]
                wb1 = wc1[lane]

                def body(j, _):
                    for u in range(4):
                        sl = pl.ds(j * 64 + u * 16, 16)
                        b0[rr, sl] = wb0 * b0[rr, sl] + wb1 * b1[rr, sl]
                    return 0
                lax.fori_loop(0, D // 64, body, 0)
            pltpu.sync_copy(b0, out_hbm.at[pl.ds(base + c * CH, CH)])

    return k(y_sorted, dest, w)


def kernel(hidden_states, gate_weight, gate_up_weights, down_weights):
    x = hidden_states.reshape(-1, D)
    dest, w, off = _route(x, gate_weight)
    meta = _tile_metadata(off)
    x_sorted = _sc_dispatch(x, dest)
    y = _gmm(x_sorted, gate_up_weights, down_weights, meta)
    out = _sc_combine(y, dest, w)
    return out.reshape(hidden_states.shape)


# speedup vs baseline: 1.8093x; 1.1073x over previous
import functools

import jax
import jax.numpy as jnp
from jax import lax
from jax.experimental import pallas as pl
from jax.experimental.pallas import tpu as pltpu
from jax.experimental.pallas import tpu_sc as plsc

D = 2048
I = 1408
E = 8
T = 4096
A = T * 2

TM = 512
M = A // TM
NV = M + E - 1
KT = 2048
NK = D // KT

NW = 32
TPW = T // NW
CH = 16
NCH = TPW // CH


def _cumsum_lanes(a, n):
    s = 1
    while s < n:
        a = a + jnp.concatenate(
            [jnp.zeros((a.shape[0], s), a.dtype), a[:, : n - s]], axis=1)
        s *= 2
    return a


def _router_kernel(x_ref, gw_ref, dest_ref, w_ref, offs_ref, logits_scr):
    step = pl.program_id(0)
    nb = pl.num_programs(0)
    xb = x_ref[...]
    lg = lax.dot_general(gw_ref[...], xb, (((1,), (1,)), ((), ())),
                         preferred_element_type=jnp.float32)
    tb = xb.shape[0]
    logits_scr[:, pl.ds(step * tb, tb)] = lg

    @pl.when(step == nb - 1)
    def _():
        lt = logits_scr[...]
        m = jnp.max(lt, axis=0, keepdims=True)
        exl = jnp.exp(lt - m)
        p = exl / jnp.sum(exl, axis=0, keepdims=True)
        ei = lax.broadcasted_iota(jnp.int32, (E, T), 0)
        m0 = jnp.max(p, axis=0, keepdims=True)
        e0 = jnp.min(jnp.where(p == m0, ei, E), axis=0, keepdims=True)
        is0 = ei == e0
        p2 = jnp.where(is0, -1.0, p)
        m1 = jnp.max(p2, axis=0, keepdims=True)
        e1 = jnp.min(jnp.where(p2 == m1, ei, E), axis=0, keepdims=True)
        is1 = ei == e1
        s = m0 + m1
        s = jnp.where(s <= 0.0, 1.0, s)
        oh0 = is0.astype(jnp.int32)
        oh1 = is1.astype(jnp.int32)
        c0 = _cumsum_lanes(oh0, T)
        cnt0 = c0[:, T - 1 : T]
        c1 = _cumsum_lanes(oh1, T) + cnt0
        cnt = c1[:, T - 1 : T]
        inc = cnt
        st = 1
        while st < E:
            inc = inc + jnp.concatenate(
                [jnp.zeros((st, 1), inc.dtype), inc[: E - st, :]], axis=0)
            st *= 2
        off = inc - cnt
        dest_ref[0:1, :] = jnp.sum(oh0 * (off + c0 - 1), axis=0, keepdims=True)
        dest_ref[1:2, :] = jnp.sum(oh1 * (off + c1 - 1), axis=0, keepdims=True)
        w_ref[0:1, :] = m0 / s
        w_ref[1:2, :] = m1 / s
        offs_ref[...] = off


def _route(x, gate_weight, interpret=False):
    tb = 512
    return pl.pallas_call(
        _router_kernel,
        grid=(T // tb,),
        in_specs=[
            pl.BlockSpec((tb, D), lambda s: (s, 0)),
            pl.BlockSpec((E, D), lambda s: (0, 0)),
        ],
        out_specs=[
            pl.BlockSpec((2, T), lambda s: (0, 0)),
            pl.BlockSpec((2, T), lambda s: (0, 0)),
            pl.BlockSpec((E, 1), lambda s: (0, 0)),
        ],
        out_shape=[
            jax.ShapeDtypeStruct((2, T), jnp.int32),
            jax.ShapeDtypeStruct((2, T), jnp.float32),
            jax.ShapeDtypeStruct((E, 1), jnp.int32),
        ],
        scratch_shapes=[pltpu.VMEM((E, T), jnp.float32)],
        interpret=interpret,
    )(x, gate_weight)


def _tile_metadata(off):
    off_e = off[:, 0]
    seg_lo = off_e
    seg_hi = jnp.concatenate([off_e[1:], jnp.array([A], jnp.int32)])
    mgrid = jnp.arange(M, dtype=jnp.int32)[None, :]
    st = jnp.maximum(seg_lo[:, None], mgrid * TM)
    en = jnp.minimum(seg_hi[:, None], mgrid * TM + TM)
    needed = en > st
    flat = needed.reshape(-1)
    rank = jnp.cumsum(flat) - flat.astype(jnp.int32)
    vidx = jnp.where(flat, rank, NV)
    incol = jnp.cumsum(needed, axis=0) - needed.astype(jnp.int32)
    first_tile = (needed & (incol == 0)).reshape(-1).astype(jnp.int32)
    inrow = jnp.cumsum(needed, axis=1) - needed.astype(jnp.int32)
    first_of_expert = (needed & (inrow == 0)).reshape(-1).astype(jnp.int32)
    mt = jnp.full((NV,), M - 1, jnp.int32).at[vidx].set(
        jnp.broadcast_to(mgrid, (E, M)).reshape(-1), mode="drop")
    gid = jnp.full((NV,), E - 1, jnp.int32).at[vidx].set(
        jnp.broadcast_to(jnp.arange(E, dtype=jnp.int32)[:, None],
                         (E, M)).reshape(-1), mode="drop")
    rs = jnp.zeros((NV,), jnp.int32).at[vidx].set(st.reshape(-1), mode="drop")
    re_ = jnp.zeros((NV,), jnp.int32).at[vidx].set(en.reshape(-1), mode="drop")
    fs = jnp.zeros((NV,), jnp.int32).at[vidx].set(first_tile, mode="drop")
    enew = jnp.zeros((NV,), jnp.int32).at[vidx].set(first_of_expert,
                                                   mode="drop")
    return mt, gid, rs, re_, fs, enew


NC = 4
DC = D // NC
IC = I // NC


def _gmm_kernel(mt, gid, rs, re_, fs, enew,
                x_ref, gu_ref, dn_ref, o_ref, g_s, u_s, d_s):
    v = pl.program_id(0)
    c = pl.program_id(1)

    @pl.when(enew[v] == 1)
    def _():
        blk = gu_ref[0]
        g_s[pl.ds(c * DC, DC), :] = blk[:, :I].astype(jnp.bfloat16)
        u_s[pl.ds(c * DC, DC), :] = blk[:, I:].astype(jnp.bfloat16)
        d_s[pl.ds(c * IC, IC), :] = dn_ref[0].astype(jnp.bfloat16)

    @pl.when(c == NC - 1)
    def _():
        xb = x_ref[...].astype(jnp.bfloat16)
        g = jnp.dot(xb, g_s[...], preferred_element_type=jnp.float32)
        u = jnp.dot(xb, u_s[...], preferred_element_type=jnp.float32)
        act = g * jax.nn.sigmoid(g) * u
        r = mt[v] * TM + lax.broadcasted_iota(jnp.int32, (TM, 1), 0)
        rm = (r >= rs[v]) & (r < re_[v])
        act = jnp.where(rm, act, 0.0).astype(jnp.bfloat16)
        contrib = jnp.dot(act, d_s[...], preferred_element_type=jnp.float32)

        @pl.when(fs[v] == 1)
        def _():
            o_ref[...] = contrib

        @pl.when(fs[v] != 1)
        def _():
            o_ref[...] += contrib


def _gmm(x_sorted, gate_up_w, down_w, meta, interpret=False):
    def wchunk(c, enew, v):
        return enew[v] * c + (1 - enew[v]) * (NC - 1)

    grid_spec = pltpu.PrefetchScalarGridSpec(
        num_scalar_prefetch=6,
        grid=(NV, NC),
        in_specs=[
            pl.BlockSpec((TM, D),
                         lambda v, c, mt, gid, rs, re_, fs, en: (mt[v], 0)),
            pl.BlockSpec((1, DC, 2 * I),
                         lambda v, c, mt, gid, rs, re_, fs, en:
                         (gid[v], wchunk(c, en, v), 0)),
            pl.BlockSpec((1, IC, D),
                         lambda v, c, mt, gid, rs, re_, fs, en:
                         (gid[v], wchunk(c, en, v), 0)),
        ],
        out_specs=pl.BlockSpec((TM, D),
                               lambda v, c, mt, gid, rs, re_, fs, en:
                               (mt[v], 0)),
        scratch_shapes=[pltpu.VMEM((D, I), jnp.bfloat16),
                        pltpu.VMEM((D, I), jnp.bfloat16),
                        pltpu.VMEM((I, D), jnp.bfloat16)],
    )
    return pl.pallas_call(
        _gmm_kernel,
        grid_spec=grid_spec,
        out_shape=jax.ShapeDtypeStruct((A, D), jnp.float32),
        interpret=interpret,
    )(*meta, x_sorted, gate_up_w, down_w)


def _sc_dispatch(x, dest):
    mesh = plsc.VectorSubcoreMesh(core_axis_name="c", subcore_axis_name="s")

    @functools.partial(
        pl.kernel, mesh=mesh,
        out_type=jax.ShapeDtypeStruct((A, D), jnp.float32),
        scratch_types=[
            pltpu.VMEM((2, TPW), jnp.int32),
            pltpu.VMEM((CH, D), jnp.float32),
            pltpu.VMEM((CH, D), jnp.float32),
            pltpu.SemaphoreType.DMA,
            pltpu.SemaphoreType.DMA,
        ],
    )
    def k(x_hbm, dest_hbm, xs_hbm, d_v, xbuf0, xbuf1, sem_l, sem_s):
        cid = lax.axis_index("c")
        sid = lax.axis_index("s")
        wid = sid * 2 + cid
        base = wid * TPW
        pltpu.sync_copy(dest_hbm.at[:, pl.ds(base, TPW)], d_v)
        bufs = (xbuf0, xbuf1)
        loads = [None] * NCH
        scats = [None] * NCH
        loads[0] = pltpu.async_copy(x_hbm.at[pl.ds(base, CH)], bufs[0], sem_l)
        for c in range(NCH):
            cur = bufs[c % 2]
            if c + 1 < NCH:
                if c >= 1:
                    scats[c - 1][0].wait()
                    scats[c - 1][1].wait()
                loads[c + 1] = pltpu.async_copy(
                    x_hbm.at[pl.ds(base + (c + 1) * CH, CH)],
                    bufs[(c + 1) % 2], sem_l)
            loads[c].wait()
            i0 = d_v[0, pl.ds(c * CH, CH)]
            i1 = d_v[1, pl.ds(c * CH, CH)]
            scats[c] = (pltpu.async_copy(cur, xs_hbm.at[i0], sem_s),
                        pltpu.async_copy(cur, xs_hbm.at[i1], sem_s))
        for c in (NCH - 2, NCH - 1):
            scats[c][0].wait()
            scats[c][1].wait()

    return k(x, dest)


def _sc_combine(y_sorted, dest, w):
    mesh = plsc.VectorSubcoreMesh(core_axis_name="c", subcore_axis_name="s")

    @functools.partial(
        pl.kernel, mesh=mesh,
        out_type=jax.ShapeDtypeStruct((T, D), jnp.float32),
        scratch_types=[
            pltpu.VMEM((2, TPW), jnp.int32),
            pltpu.VMEM((2, TPW), jnp.float32),
            pltpu.VMEM((CH, D), jnp.float32),
            pltpu.VMEM((CH, D), jnp.float32),
            pltpu.VMEM((CH, D), jnp.float32),
            pltpu.SemaphoreType.DMA,
            pltpu.SemaphoreType.DMA,
        ],
    )
    def k(ys_hbm, dest_hbm, w_hbm, out_hbm, d_v, w_v, b0, b1, bo,
          sem_g, sem_o):
        cid = lax.axis_index("c")
        sid = lax.axis_index("s")
        wid = sid * 2 + cid
        base = wid * TPW
        pltpu.sync_copy(dest_hbm.at[:, pl.ds(base, TPW)], d_v)
        pltpu.sync_copy(w_hbm.at[:, pl.ds(base, TPW)], w_v)
        gats = [None] * NCH
        outs = [None] * NCH
        gats[0] = (pltpu.async_copy(ys_hbm.at[d_v[0, pl.ds(0, CH)]], b0, sem_g),
                   pltpu.async_copy(ys_hbm.at[d_v[1, pl.ds(0, CH)]], b1, sem_g))
        for c in range(NCH):
            gats[c][0].wait()
            gats[c][1].wait()
            if c >= 1:
                outs[c - 1].wait()
            wc0 = w_v[0, pl.ds(c * CH, CH)]
            wc1 = w_v[1, pl.ds(c * CH, CH)]
            for rr in range(CH):
                lane = jnp.full((CH,), rr, jnp.int32)
                wb0 = wc0[lane]
                wb1 = wc1[lane]

                def body(j, _):
                    for u in range(8):
                        sl = pl.ds(j * 128 + u * 16, 16)
                        bo[rr, sl] = wb0 * b0[rr, sl] + wb1 * b1[rr, sl]
                    return 0
                lax.fori_loop(0, D // 128, body, 0)
            outs[c] = pltpu.async_copy(
                bo, out_hbm.at[pl.ds(base + c * CH, CH)], sem_o)
            if c + 1 < NCH:
                gats[c + 1] = (
                    pltpu.async_copy(ys_hbm.at[d_v[0, pl.ds((c + 1) * CH, CH)]],
                                     b0, sem_g),
                    pltpu.async_copy(ys_hbm.at[d_v[1, pl.ds((c + 1) * CH, CH)]],
                                     b1, sem_g))
        outs[NCH - 1].wait()

    return k(y_sorted, dest, w)


def kernel(hidden_states, gate_weight, gate_up_weights, down_weights):
    x = hidden_states.reshape(-1, D)
    dest, w, off = _route(x, gate_weight)
    meta = _tile_metadata(off)
    x_sorted = _sc_dispatch(x, dest)
    y = _gmm(x_sorted, gate_up_weights, down_weights, meta)
    out = _sc_combine(y, dest, w)
    return out.reshape(hidden_states.shape)
